# baseline (speedup 1.0000x reference)
import functools

import jax
import jax.numpy as jnp
from jax import lax
from jax.experimental import pallas as pl
from jax.experimental.pallas import tpu as pltpu
from jax.experimental.pallas import tpu_sc as plsc

VOCAB = 100000
EMBED = 64
HIDDEN = 256
BATCH = 16
SEQ = 50
NTOK = BATCH * SEQ
FC_TILE = 4096
FC_GRID = (VOCAB + FC_TILE - 1) // FC_TILE

_SC_CORES = 2

_CDIMS = (((1,), (1,)), ((), ()))


def _sc_gather(emb, idx):
    mesh = plsc.ScalarSubcoreMesh(axis_name="core", num_cores=_SC_CORES)
    per_core = NTOK // _SC_CORES
    idx_pad = 896
    burst = 16

    @functools.partial(
        pl.kernel,
        mesh=mesh,
        out_type=jax.ShapeDtypeStruct((NTOK, EMBED), jnp.float32),
        scratch_types=[
            pltpu.SMEM((idx_pad,), jnp.int32),
            pltpu.SemaphoreType.DMA,
            pltpu.SemaphoreType.DMA,
        ],
    )
    def gather_kernel(table_hbm, idx_hbm, out_hbm, idx_s, isem, rsem):
        cid = lax.axis_index("core")
        base = cid * per_core
        pltpu.sync_copy(idx_hbm, idx_s)

        def fire(i0):
            for j in range(burst):
                r = idx_s[base + i0 + j]
                pltpu.make_async_copy(
                    table_hbm.at[pl.ds(r, 1), :],
                    out_hbm.at[pl.ds(base + i0 + j, 1), :], rsem).start()

        def drain(i0):
            for j in range(burst):
                pltpu.make_async_copy(
                    table_hbm.at[pl.ds(0, 1), :],
                    out_hbm.at[pl.ds(base + i0 + j, 1), :], rsem).wait()

        fire(0)
        fire(burst)

        @pl.loop(2 * burst, per_core, step=burst)
        def _(i0):
            fire(i0)
            drain(i0 - 2 * burst)

        drain(per_core - 2 * burst)
        drain(per_core - burst)

    idx2 = jnp.concatenate(
        [idx, jnp.zeros((idx_pad - NTOK,), jnp.int32)])
    return gather_kernel(emb, idx2)


def _fused_body(e_ref, wih0h_ref, wih0l_ref, whh0h_ref, whh0l_ref, b0_ref,
                wih1h_ref, wih1l_ref, whh1h_ref, whh1l_ref, b1_ref, h0_ref,
                c0_ref, fcw_ref, fcb_ref, out_ref, hn_ref, cn_ref, g_ref,
                y1_ref, yb_ref):

    def dot3(a, w_hi, w_lo):
        a_hi = a.astype(jnp.bfloat16)
        a_lo = (a - a_hi.astype(jnp.float32)).astype(jnp.bfloat16)
        return (jnp.dot(a_hi, w_hi, preferred_element_type=jnp.float32)
                + jnp.dot(a_hi, w_lo, preferred_element_type=jnp.float32)
                + jnp.dot(a_lo, w_hi, preferred_element_type=jnp.float32))

    @pl.when(pl.program_id(0) == 0)
    def _lstm():
        g_ref[...] = dot3(e_ref[...], wih0h_ref[...], wih0l_ref[...]) \
            + b0_ref[...]

        def gate_stack(gates, c):
            ig = jax.nn.sigmoid(gates[:, 0 * HIDDEN:1 * HIDDEN])
            fg = jax.nn.sigmoid(gates[:, 1 * HIDDEN:2 * HIDDEN])
            gg = jnp.tanh(gates[:, 2 * HIDDEN:3 * HIDDEN])
            og = jax.nn.sigmoid(gates[:, 3 * HIDDEN:4 * HIDDEN])
            c_new = fg * c + ig * gg
            return og * jnp.tanh(c_new), c_new

        def step(t, carry):
            h0, c0, h1, c1 = carry
            t0 = jnp.where(t < SEQ, t, 0)
            gates0 = g_ref[pl.ds(t0 * BATCH, BATCH), :] + dot3(
                h0, whh0h_ref[...], whh0l_ref[...])
            h0n, c0n = gate_stack(gates0, c0)
            gates1 = (dot3(h0, wih1h_ref[...], wih1l_ref[...]) + b1_ref[...]
                      + dot3(h1, whh1h_ref[...], whh1l_ref[...]))
            h1n, c1n = gate_stack(gates1, c1)

            t1 = jnp.where(t > 0, t - 1, 0)

            @pl.when(t > 0)
            def _():
                y1_ref[pl.ds(t1 * BATCH, BATCH), :] = h1n

            keep0 = t < SEQ
            keep1 = t > 0
            return (jnp.where(keep0, h0n, h0), jnp.where(keep0, c0n, c0),
                    jnp.where(keep1, h1n, h1), jnp.where(keep1, c1n, c1))

        h0, c0, h1, c1 = lax.fori_loop(
            0, SEQ + 1, step, (h0_ref[0], c0_ref[0], h0_ref[1], c0_ref[1]))
        hn_ref[0] = h0
        cn_ref[0] = c0
        hn_ref[1] = h1
        cn_ref[1] = c1

        yb_ref[...] = jnp.transpose(
            y1_ref[...].reshape(SEQ, BATCH, HIDDEN), (1, 0, 2)).reshape(
                NTOK, HIDDEN)

    out_ref[...] = lax.dot_general(
        yb_ref[...], fcw_ref[...], _CDIMS,
        preferred_element_type=jnp.float32) + fcb_ref[...]


def _fused_call(e, wih0h, wih0l, whh0h, whh0l, b0, wih1h, wih1l, whh1h,
                whh1l, b1, h0, c0, fc_w, fc_b2d):
    const = lambda i: (0, 0)
    return pl.pallas_call(
        _fused_body,
        grid=(FC_GRID,),
        in_specs=[
            pl.BlockSpec((NTOK, EMBED), const),
            pl.BlockSpec((EMBED, 4 * HIDDEN), const),
            pl.BlockSpec((EMBED, 4 * HIDDEN), const),
            pl.BlockSpec((HIDDEN, 4 * HIDDEN), const),
            pl.BlockSpec((HIDDEN, 4 * HIDDEN), const),
            pl.BlockSpec((1, 4 * HIDDEN), const),
            pl.BlockSpec((HIDDEN, 4 * HIDDEN), const),
            pl.BlockSpec((HIDDEN, 4 * HIDDEN), const),
            pl.BlockSpec((HIDDEN, 4 * HIDDEN), const),
            pl.BlockSpec((HIDDEN, 4 * HIDDEN), const),
            pl.BlockSpec((1, 4 * HIDDEN), const),
            pl.BlockSpec((2, BATCH, HIDDEN), lambda i: (0, 0, 0)),
            pl.BlockSpec((2, BATCH, HIDDEN), lambda i: (0, 0, 0)),
            pl.BlockSpec((FC_TILE, HIDDEN), lambda i: (i, 0)),
            pl.BlockSpec((1, FC_TILE), lambda i: (0, i)),
        ],
        out_specs=(
            pl.BlockSpec((NTOK, FC_TILE), lambda i: (0, i)),
            pl.BlockSpec((2, BATCH, HIDDEN), lambda i: (0, 0, 0)),
            pl.BlockSpec((2, BATCH, HIDDEN), lambda i: (0, 0, 0)),
        ),
        out_shape=(
            jax.ShapeDtypeStruct((NTOK, VOCAB), jnp.float32),
            jax.ShapeDtypeStruct((2, BATCH, HIDDEN), jnp.float32),
            jax.ShapeDtypeStruct((2, BATCH, HIDDEN), jnp.float32),
        ),
        scratch_shapes=[
            pltpu.VMEM((NTOK, 4 * HIDDEN), jnp.float32),
            pltpu.VMEM((NTOK, HIDDEN), jnp.float32),
            pltpu.VMEM((NTOK, HIDDEN), jnp.float32),
        ],
        compiler_params=pltpu.CompilerParams(
            dimension_semantics=("arbitrary",)),
    )(e, wih0h, wih0l, whh0h, whh0l, b0, wih1h, wih1l, whh1h, whh1l, b1,
      h0, c0, fc_w, fc_b2d)


def kernel(x, h0, c0, emb, w_ih0, w_hh0, b_ih0, b_hh0, w_ih1, w_hh1, b_ih1,
           b_hh1, fc_w, fc_b):
    idx = jnp.transpose(x).reshape(-1)

    e = _sc_gather(emb, idx)

    b0 = (b_ih0 + b_hh0).reshape(1, 4 * HIDDEN)
    b1 = (b_ih1 + b_hh1).reshape(1, 4 * HIDDEN)

    def split_bf16(w):
        wt = jnp.transpose(w)
        w_hi = wt.astype(jnp.bfloat16)
        w_lo = (wt - w_hi.astype(jnp.float32)).astype(jnp.bfloat16)
        return w_hi, w_lo

    wih0h, wih0l = split_bf16(w_ih0)
    whh0h, whh0l = split_bf16(w_hh0)
    wih1h, wih1l = split_bf16(w_ih1)
    whh1h, whh1l = split_bf16(w_hh1)
    out, hN, cN = _fused_call(
        e, wih0h, wih0l, whh0h, whh0l, b0, wih1h, wih1l, whh1h, whh1l,
        b1, h0, c0, fc_w, fc_b.reshape(1, VOCAB))
    return out, hN, cN


# speedup vs baseline: 1.0117x; 1.0117x over previous
import functools

import jax
import jax.numpy as jnp
from jax import lax
from jax.experimental import pallas as pl
from jax.experimental.pallas import tpu as pltpu
from jax.experimental.pallas import tpu_sc as plsc

VOCAB = 100000
EMBED = 64
HIDDEN = 256
BATCH = 16
SEQ = 50
NTOK = BATCH * SEQ
FC_TILE = 4096
FC_GRID = (VOCAB + FC_TILE - 1) // FC_TILE

_SC_CORES = 2

_CDIMS = (((1,), (1,)), ((), ()))


def _sc_gather(emb, idx):
    mesh = plsc.ScalarSubcoreMesh(axis_name="core", num_cores=_SC_CORES)
    per_core = NTOK // _SC_CORES
    idx_pad = 896
    burst = 16

    @functools.partial(
        pl.kernel,
        mesh=mesh,
        out_type=jax.ShapeDtypeStruct((NTOK, EMBED), jnp.float32),
        scratch_types=[
            pltpu.SMEM((idx_pad,), jnp.int32),
            pltpu.SemaphoreType.DMA,
            pltpu.SemaphoreType.DMA,
        ],
    )
    def gather_kernel(table_hbm, idx_hbm, out_hbm, idx_s, isem, rsem):
        cid = lax.axis_index("core")
        base = cid * per_core
        pltpu.sync_copy(idx_hbm, idx_s)

        def fire(i0):
            for j in range(burst):
                r = idx_s[base + i0 + j]
                pltpu.make_async_copy(
                    table_hbm.at[pl.ds(r, 1), :],
                    out_hbm.at[pl.ds(base + i0 + j, 1), :], rsem).start()

        def drain(i0):
            for j in range(burst):
                pltpu.make_async_copy(
                    table_hbm.at[pl.ds(0, 1), :],
                    out_hbm.at[pl.ds(base + i0 + j, 1), :], rsem).wait()

        fire(0)
        fire(burst)

        @pl.loop(2 * burst, per_core, step=burst)
        def _(i0):
            fire(i0)
            drain(i0 - 2 * burst)

        drain(per_core - 2 * burst)
        drain(per_core - burst)

    idx2 = jnp.concatenate(
        [idx, jnp.zeros((idx_pad - NTOK,), jnp.int32)])
    return gather_kernel(emb, idx2)


def _fused_body(e_ref, wih0h_ref, wih0l_ref, whh0h_ref, whh0l_ref, b0_ref,
                wih1h_ref, wih1l_ref, whh1h_ref, whh1l_ref, b1_ref, h0_ref,
                c0_ref, fcw_ref, fcb_ref, out_ref, hn_ref, cn_ref, g_ref,
                y1_ref, yb_ref):

    def dot3(a, w_hi, w_lo):
        a_hi = a.astype(jnp.bfloat16)
        a_lo = (a - a_hi.astype(jnp.float32)).astype(jnp.bfloat16)
        return (jnp.dot(a_hi, w_hi, preferred_element_type=jnp.float32)
                + jnp.dot(a_hi, w_lo, preferred_element_type=jnp.float32)
                + jnp.dot(a_lo, w_hi, preferred_element_type=jnp.float32))

    @pl.when(pl.program_id(0) == 0)
    def _lstm():
        def run_layer(gates_all, whh_h_ref, whh_l_ref, h_init, c_init,
                      out_vmem):
            def step(t, carry):
                h, c = carry
                h_hi = h.astype(jnp.bfloat16)
                h_lo = (h - h_hi.astype(jnp.float32)).astype(jnp.bfloat16)
                w_h = whh_h_ref[...]
                hw = (jnp.dot(h_hi, w_h,
                              preferred_element_type=jnp.float32)
                      + jnp.dot(h_hi, whh_l_ref[...],
                                preferred_element_type=jnp.float32)
                      + jnp.dot(h_lo, w_h,
                                preferred_element_type=jnp.float32))
                gates = g_ref[pl.ds(t * BATCH, BATCH), :] + hw
                ig = jax.nn.sigmoid(gates[:, 0 * HIDDEN:1 * HIDDEN])
                fg = jax.nn.sigmoid(gates[:, 1 * HIDDEN:2 * HIDDEN])
                gg = jnp.tanh(gates[:, 2 * HIDDEN:3 * HIDDEN])
                og = jax.nn.sigmoid(gates[:, 3 * HIDDEN:4 * HIDDEN])
                c_new = fg * c + ig * gg
                h_new = og * jnp.tanh(c_new)
                out_vmem[pl.ds(t * BATCH, BATCH), :] = h_new
                return h_new, c_new

            g_ref[...] = gates_all
            return lax.fori_loop(0, SEQ, step, (h_init, c_init))

        gates0 = dot3(e_ref[...], wih0h_ref[...], wih0l_ref[...]) \
            + b0_ref[...]
        h, c = run_layer(gates0, whh0h_ref, whh0l_ref, h0_ref[0],
                         c0_ref[0], yb_ref)
        hn_ref[0] = h
        cn_ref[0] = c

        gates1 = dot3(yb_ref[...], wih1h_ref[...], wih1l_ref[...]) \
            + b1_ref[...]
        h, c = run_layer(gates1, whh1h_ref, whh1l_ref, h0_ref[1],
                         c0_ref[1], y1_ref)
        hn_ref[1] = h
        cn_ref[1] = c

        yb_ref[...] = jnp.transpose(
            y1_ref[...].reshape(SEQ, BATCH, HIDDEN), (1, 0, 2)).reshape(
                NTOK, HIDDEN)

    out_ref[...] = lax.dot_general(
        yb_ref[...], fcw_ref[...], _CDIMS,
        preferred_element_type=jnp.float32) + fcb_ref[...]


def _fused_call(e, wih0h, wih0l, whh0h, whh0l, b0, wih1h, wih1l, whh1h,
                whh1l, b1, h0, c0, fc_w, fc_b2d):
    const = lambda i: (0, 0)
    return pl.pallas_call(
        _fused_body,
        grid=(FC_GRID,),
        in_specs=[
            pl.BlockSpec((NTOK, EMBED), const),
            pl.BlockSpec((EMBED, 4 * HIDDEN), const),
            pl.BlockSpec((EMBED, 4 * HIDDEN), const),
            pl.BlockSpec((HIDDEN, 4 * HIDDEN), const),
            pl.BlockSpec((HIDDEN, 4 * HIDDEN), const),
            pl.BlockSpec((1, 4 * HIDDEN), const),
            pl.BlockSpec((HIDDEN, 4 * HIDDEN), const),
            pl.BlockSpec((HIDDEN, 4 * HIDDEN), const),
            pl.BlockSpec((HIDDEN, 4 * HIDDEN), const),
            pl.BlockSpec((HIDDEN, 4 * HIDDEN), const),
            pl.BlockSpec((1, 4 * HIDDEN), const),
            pl.BlockSpec((2, BATCH, HIDDEN), lambda i: (0, 0, 0)),
            pl.BlockSpec((2, BATCH, HIDDEN), lambda i: (0, 0, 0)),
            pl.BlockSpec((FC_TILE, HIDDEN), lambda i: (i, 0)),
            pl.BlockSpec((1, FC_TILE), lambda i: (0, i)),
        ],
        out_specs=(
            pl.BlockSpec((NTOK, FC_TILE), lambda i: (0, i)),
            pl.BlockSpec((2, BATCH, HIDDEN), lambda i: (0, 0, 0)),
            pl.BlockSpec((2, BATCH, HIDDEN), lambda i: (0, 0, 0)),
        ),
        out_shape=(
            jax.ShapeDtypeStruct((NTOK, VOCAB), jnp.float32),
            jax.ShapeDtypeStruct((2, BATCH, HIDDEN), jnp.float32),
            jax.ShapeDtypeStruct((2, BATCH, HIDDEN), jnp.float32),
        ),
        scratch_shapes=[
            pltpu.VMEM((NTOK, 4 * HIDDEN), jnp.float32),
            pltpu.VMEM((NTOK, HIDDEN), jnp.float32),
            pltpu.VMEM((NTOK, HIDDEN), jnp.float32),
        ],
        compiler_params=pltpu.CompilerParams(
            dimension_semantics=("arbitrary",)),
    )(e, wih0h, wih0l, whh0h, whh0l, b0, wih1h, wih1l, whh1h, whh1l, b1,
      h0, c0, fc_w, fc_b2d)


def kernel(x, h0, c0, emb, w_ih0, w_hh0, b_ih0, b_hh0, w_ih1, w_hh1, b_ih1,
           b_hh1, fc_w, fc_b):
    idx = jnp.transpose(x).reshape(-1)

    e = _sc_gather(emb, idx)

    b0 = (b_ih0 + b_hh0).reshape(1, 4 * HIDDEN)
    b1 = (b_ih1 + b_hh1).reshape(1, 4 * HIDDEN)

    def split_bf16(w):
        wt = jnp.transpose(w)
        w_hi = wt.astype(jnp.bfloat16)
        w_lo = (wt - w_hi.astype(jnp.float32)).astype(jnp.bfloat16)
        return w_hi, w_lo

    wih0h, wih0l = split_bf16(w_ih0)
    whh0h, whh0l = split_bf16(w_hh0)
    wih1h, wih1l = split_bf16(w_ih1)
    whh1h, whh1l = split_bf16(w_hh1)
    out, hN, cN = _fused_call(
        e, wih0h, wih0l, whh0h, whh0l, b0, wih1h, wih1l, whh1h, whh1l,
        b1, h0, c0, fc_w, fc_b.reshape(1, VOCAB))
    return out, hN, cN
